# baseline (speedup 1.0000x reference)
import math

import jax
import jax.numpy as jnp
import numpy as np
from jax import lax
from jax.experimental import pallas as pl
from jax.experimental.pallas import tpu as pltpu
from jax.experimental.pallas import tpu_sc as plsc

VOCAB_SIZE = 1000000
DIM = 64
SEQ_LEN = 200
BATCH = 1024

NUM_WORKERS = 32
TOTAL_ROWS = BATCH * SEQ_LEN
ROWS_PER_WORKER = TOTAL_ROWS // NUM_WORKERS
CHUNK = 128
N_CHUNKS = ROWS_PER_WORKER // CHUNK
NBUF = 10
D_PE = 3
D_WR = 6


def _position_embedding_np():
    i = np.arange(SEQ_LEN, dtype=np.float64)[:, None]
    j = np.arange(DIM, dtype=np.float64)[None, :]
    even_mask = (np.arange(DIM) % 2 == 0)[None, :]
    temp_even = np.exp(-(j / DIM) * math.log(10000.0))
    temp_odd = np.exp(-((j - 1.0) / DIM) * math.log(10000.0))
    pe = np.where(even_mask, np.sin(i * temp_even), np.cos(i * temp_odd))
    return pe.astype(np.float32)


_PE = _position_embedding_np()

_POS = ((np.arange(N_CHUNKS * CHUNK) % SEQ_LEN)
        .astype(np.int32).reshape(N_CHUNKS, CHUNK))


def _sc_body(idx_hbm, pos_hbm, table_hbm, pe_hbm, out_hbm,
             idx_v, pos_v, rows_v, gsem, psem, wsem):
    wid = lax.axis_index("s") * 2 + lax.axis_index("c")
    wstart = wid * ROWS_PER_WORKER

    pltpu.sync_copy(idx_hbm.at[wid], idx_v)
    pltpu.sync_copy(pos_hbm, pos_v)

    def slot(g):
        return lax.rem(g, NBUF)

    def fire_tbl(g):
        pltpu.async_copy(table_hbm.at[idx_v.at[g]], rows_v.at[slot(g)],
                         gsem.at[slot(g)])

    def wait_tbl(g):
        pltpu.make_async_copy(table_hbm.at[idx_v.at[g]], rows_v.at[slot(g)],
                              gsem.at[slot(g)]).wait()

    def fire_pe(g):
        pltpu.async_copy(pe_hbm.at[pos_v.at[g]], rows_v.at[slot(g)],
                         psem.at[slot(g)], add=True)

    def wait_pe(g):
        pltpu.make_async_copy(pe_hbm.at[pos_v.at[g]], rows_v.at[slot(g)],
                              psem.at[slot(g)]).wait()

    def fire_write(g):
        pltpu.async_copy(rows_v.at[slot(g)],
                         out_hbm.at[pl.ds(wstart + g * CHUNK, CHUNK)],
                         wsem.at[slot(g)])

    def wait_write(g):
        pltpu.make_async_copy(rows_v.at[slot(g)],
                              out_hbm.at[pl.ds(wstart + g * CHUNK, CHUNK)],
                              wsem.at[slot(g)]).wait()

    for i in range(NBUF):
        fire_tbl(i)
        if i >= D_PE:
            wait_tbl(i - D_PE)
            fire_pe(i - D_PE)
        if i >= D_WR:
            wait_pe(i - D_WR)
            fire_write(i - D_WR)

    def steady(i, carry):
        wait_write(i - NBUF)
        fire_tbl(i)
        wait_tbl(i - D_PE)
        fire_pe(i - D_PE)
        wait_pe(i - D_WR)
        fire_write(i - D_WR)
        return carry

    lax.fori_loop(NBUF, N_CHUNKS, steady, 0)

    for g in range(N_CHUNKS - D_PE, N_CHUNKS):
        wait_tbl(g)
        fire_pe(g)
    for g in range(N_CHUNKS - D_WR, N_CHUNKS):
        wait_pe(g)
        fire_write(g)
    for g in range(N_CHUNKS - NBUF, N_CHUNKS):
        wait_write(g)


TCOLS = 2048


def _tc_transpose_body(tt_ref, out_ref):
    blk_t = tt_ref[...].T.reshape(TCOLS // 2, 2, DIM)
    out_ref[:, 0:DIM] = blk_t[:, 0, :]
    out_ref[:, DIM:2 * DIM] = blk_t[:, 1, :]


def _tc_transpose(tt):
    return pl.pallas_call(
        _tc_transpose_body,
        grid=((VOCAB_SIZE + TCOLS - 1) // TCOLS,),
        in_specs=[pl.BlockSpec((DIM, TCOLS), lambda i: (0, i))],
        out_specs=pl.BlockSpec((TCOLS // 2, 2 * DIM), lambda i: (i, 0)),
        out_shape=jax.ShapeDtypeStruct((VOCAB_SIZE // 2, 2 * DIM), jnp.float32),
    )(tt)


@jax.jit
def kernel(x, table):
    table_rows = _tc_transpose(table.T).reshape(VOCAB_SIZE, DIM)
    idx = x.reshape(NUM_WORKERS, N_CHUNKS, CHUNK)
    pe = jnp.asarray(_PE)
    pos = jnp.asarray(_POS)
    run = pl.kernel(
        _sc_body,
        out_type=jax.ShapeDtypeStruct((TOTAL_ROWS, DIM), jnp.float32),
        mesh=plsc.VectorSubcoreMesh(core_axis_name="c", subcore_axis_name="s"),
        scratch_types=[
            pltpu.VMEM((N_CHUNKS, CHUNK), jnp.int32),
            pltpu.VMEM((N_CHUNKS, CHUNK), jnp.int32),
            pltpu.VMEM((NBUF, CHUNK, DIM), jnp.float32),
            pltpu.SemaphoreType.DMA((NBUF,)),
            pltpu.SemaphoreType.DMA((NBUF,)),
            pltpu.SemaphoreType.DMA((NBUF,)),
        ],
        compiler_params=pltpu.CompilerParams(use_tc_tiling_on_sc=False),
    )
    out = run(idx, pos, table_rows, pe)
    return out.reshape(BATCH, SEQ_LEN, DIM)


# speedup vs baseline: 1.3067x; 1.3067x over previous
import math

import jax
import jax.numpy as jnp
import numpy as np
from jax import lax
from jax.experimental import pallas as pl
from jax.experimental.pallas import tpu as pltpu
from jax.experimental.pallas import tpu_sc as plsc

VOCAB_SIZE = 1000000
DIM = 64
SEQ_LEN = 200
BATCH = 1024

NUM_WORKERS = 32
TOTAL_ROWS = BATCH * SEQ_LEN
ROWS_PER_WORKER = TOTAL_ROWS // NUM_WORKERS
CHUNK = 128
N_CHUNKS = ROWS_PER_WORKER // CHUNK
NBUF = 10
D_PE = 3
D_WR = 6


def _position_embedding_np():
    i = np.arange(SEQ_LEN, dtype=np.float64)[:, None]
    j = np.arange(DIM, dtype=np.float64)[None, :]
    even_mask = (np.arange(DIM) % 2 == 0)[None, :]
    temp_even = np.exp(-(j / DIM) * math.log(10000.0))
    temp_odd = np.exp(-((j - 1.0) / DIM) * math.log(10000.0))
    pe = np.where(even_mask, np.sin(i * temp_even), np.cos(i * temp_odd))
    return pe.astype(np.float32)


_PE2 = np.concatenate([_position_embedding_np()] * 2, axis=0)


def _sc_body(idx_hbm, table_hbm, pe_hbm, out_hbm,
             idx_v, pe_v, rows_v, gsem, wsem):
    wid = lax.axis_index("s") * 2 + lax.axis_index("c")
    wstart = wid * ROWS_PER_WORKER

    pltpu.sync_copy(idx_hbm.at[wid], idx_v)
    pltpu.sync_copy(pe_hbm, pe_v)

    def slot(g):
        return lax.rem(g, NBUF)

    def fire_tbl(g):
        pltpu.async_copy(table_hbm.at[idx_v.at[g]], rows_v.at[slot(g)],
                         gsem.at[slot(g)])

    def wait_tbl(g):
        pltpu.make_async_copy(table_hbm.at[idx_v.at[g]], rows_v.at[slot(g)],
                              gsem.at[slot(g)]).wait()

    def add_pe(g):
        off = lax.rem(g * CHUNK, SEQ_LEN)
        s = slot(g)

        def row_body(r, carry):
            for q in range(DIM // 16):
                sl = pl.ds(q * 16, 16)
                rows_v[s, r, sl] += pe_v[off + r, sl]
            return carry

        lax.fori_loop(0, CHUNK, row_body, 0, unroll=8)

    def fire_write(g):
        pltpu.async_copy(rows_v.at[slot(g)],
                         out_hbm.at[pl.ds(wstart + g * CHUNK, CHUNK)],
                         wsem.at[slot(g)])

    def wait_write(g):
        pltpu.make_async_copy(rows_v.at[slot(g)],
                              out_hbm.at[pl.ds(wstart + g * CHUNK, CHUNK)],
                              wsem.at[slot(g)]).wait()

    for i in range(NBUF):
        fire_tbl(i)
        if i >= D_PE:
            wait_tbl(i - D_PE)
            add_pe(i - D_PE)
            fire_write(i - D_PE)

    def steady(i, carry):
        wait_write(i - NBUF)
        fire_tbl(i)
        wait_tbl(i - D_PE)
        add_pe(i - D_PE)
        fire_write(i - D_PE)
        return carry

    lax.fori_loop(NBUF, N_CHUNKS, steady, 0)

    for g in range(N_CHUNKS - D_PE, N_CHUNKS):
        wait_tbl(g)
        add_pe(g)
        fire_write(g)
    for g in range(N_CHUNKS - NBUF, N_CHUNKS):
        wait_write(g)


TCOLS = 8192


def _tc_transpose_body(tt_ref, out_ref):
    blk_t = tt_ref[...].T.reshape(TCOLS // 2, 2, DIM)
    out_ref[:, 0:DIM] = blk_t[:, 0, :]
    out_ref[:, DIM:2 * DIM] = blk_t[:, 1, :]


def _tc_transpose(tt):
    return pl.pallas_call(
        _tc_transpose_body,
        grid=((VOCAB_SIZE + TCOLS - 1) // TCOLS,),
        in_specs=[pl.BlockSpec((DIM, TCOLS), lambda i: (0, i))],
        out_specs=pl.BlockSpec((TCOLS // 2, 2 * DIM), lambda i: (i, 0)),
        out_shape=jax.ShapeDtypeStruct((VOCAB_SIZE // 2, 2 * DIM), jnp.float32),
    )(tt)


@jax.jit
def kernel(x, table):
    table_rows = _tc_transpose(table.T).reshape(VOCAB_SIZE, DIM)
    idx = x.reshape(NUM_WORKERS, N_CHUNKS, CHUNK)
    pe2 = jnp.asarray(_PE2)
    run = pl.kernel(
        _sc_body,
        out_type=jax.ShapeDtypeStruct((TOTAL_ROWS, DIM), jnp.float32),
        mesh=plsc.VectorSubcoreMesh(core_axis_name="c", subcore_axis_name="s"),
        scratch_types=[
            pltpu.VMEM((N_CHUNKS, CHUNK), jnp.int32),
            pltpu.VMEM((2 * SEQ_LEN, DIM), jnp.float32),
            pltpu.VMEM((NBUF, CHUNK, DIM), jnp.float32),
            pltpu.SemaphoreType.DMA((NBUF,)),
            pltpu.SemaphoreType.DMA((NBUF,)),
        ],
        compiler_params=pltpu.CompilerParams(use_tc_tiling_on_sc=False),
    )
    out = run(idx, table_rows, pe2)
    return out.reshape(BATCH, SEQ_LEN, DIM)
